# baseline (speedup 1.0000x reference)
import functools

import jax
import jax.numpy as jnp
from jax import lax
from jax.experimental import pallas as pl
from jax.experimental.pallas import tpu as pltpu

try:
    from jax.experimental.pallas import tpu_sc as plsc
except ImportError:
    plsc = None

LATENT = 256
NTOK = 8192
BN = 512
R = BN // 8


def _argmin_body(x_ref, w_ref, iiw_ref, idx_ref, x2_ref, c2_ref, ii_ref,
                 xs_ref):
    n = pl.program_id(0)

    x = x_ref[...]
    x2_ref[...] = jnp.sum(x * x, axis=1, keepdims=True).reshape(R, 8, 1)
    xs_ref[...] = x * (-2.0)

    @pl.when(n == 0)
    def _consts():
        w = w_ref[...]
        c2 = jnp.sum(w * w, axis=1)
        c2_ref[...] = jnp.broadcast_to(c2[None, :], (8, NTOK))
        ii_ref[...] = lax.broadcasted_iota(jnp.int32, (8, NTOK), 1).astype(
            jnp.float32)

    m = lax.dot_general(xs_ref[...], w_ref[...], (((1,), (1,)), ((), ())),
                        preferred_element_type=jnp.float32)
    m3 = m.reshape(R, 8, NTOK)
    d = x2_ref[...] + m3 + c2_ref[...][None]

    bmin = jnp.min(d, axis=2, keepdims=True)
    eqm = d == bmin
    onef = jnp.where(eqm, 1.0, 0.0)
    s = lax.dot_general(onef.reshape(BN, NTOK), iiw_ref[...],
                        (((1,), (0,)), ((), ())),
                        preferred_element_type=jnp.float32)
    idx_ref[...] = s[:, 0].astype(jnp.int32)

    @pl.when(jnp.max(s[:, 1]) > 1.0)
    def _ties():
        ii = ii_ref[...][None]
        bidx = jnp.min(jnp.where(eqm, ii, jnp.inf), axis=2, keepdims=True)
        idx_ref[...] = bidx.astype(jnp.int32).reshape(BN)


def _tc_argmin(flat, weight):
    n = flat.shape[0]
    iiw = jnp.stack([jnp.arange(NTOK, dtype=jnp.float32),
                     jnp.ones((NTOK,), jnp.float32)], axis=1)
    return pl.pallas_call(
        _argmin_body,
        grid=(n // BN,),
        in_specs=[
            pl.BlockSpec((BN, LATENT), lambda i: (i, 0)),
            pl.BlockSpec((NTOK, LATENT), lambda i: (0, 0)),
            pl.BlockSpec((NTOK, 2), lambda i: (0, 0)),
        ],
        out_specs=pl.BlockSpec((BN,), lambda i: (i,)),
        out_shape=jax.ShapeDtypeStruct((n,), jnp.int32),
        scratch_shapes=[
            pltpu.VMEM((R, 8, 1), jnp.float32),
            pltpu.VMEM((8, NTOK), jnp.float32),
            pltpu.VMEM((8, NTOK), jnp.float32),
            pltpu.VMEM((BN, LATENT), jnp.float32),
        ],
    )(flat, weight, iiw)


_NC, _NS = 2, 16
_NW = _NC * _NS


def _sc_gather(weight, idx):
    n = idx.shape[0]
    bpw = n // _NW

    @functools.partial(
        pl.kernel,
        mesh=plsc.VectorSubcoreMesh(core_axis_name="c", subcore_axis_name="s"),
        out_type=jax.ShapeDtypeStruct((n, LATENT), jnp.float32),
        scratch_types=[
            pltpu.VMEM((bpw,), jnp.int32),
            pltpu.VMEM((bpw, LATENT), jnp.float32),
            pltpu.SemaphoreType.DMA,
        ],
    )
    def gather_k(table_hbm, idx_hbm, out_hbm, idx_v, rows_v, sem):
        wid = lax.axis_index("s") * _NC + lax.axis_index("c")
        base = wid * bpw
        pltpu.sync_copy(idx_hbm.at[pl.ds(base, bpw)], idx_v)
        pltpu.async_copy(table_hbm.at[idx_v], rows_v, sem).wait()
        pltpu.sync_copy(rows_v, out_hbm.at[pl.ds(base, bpw)])

    return gather_k(weight, idx)


def kernel(x, weight):
    flat = x.reshape(-1, LATENT)
    idx = _tc_argmin(flat, weight)
    codes = _sc_gather(weight, idx)
    return codes.reshape(x.shape)


# speedup vs baseline: 1.6672x; 1.6672x over previous
import functools

import jax
import jax.numpy as jnp
from jax import lax
from jax.experimental import pallas as pl
from jax.experimental.pallas import tpu as pltpu

try:
    from jax.experimental.pallas import tpu_sc as plsc
except ImportError:
    plsc = None

LATENT = 256
NTOK = 8192
BN = 512
BK = 8192


R = BN // 8


def _argmin_body(x_ref, w_ref, idx_ref, x2_ref, c2_ref, ii_ref, xs_ref):
    n = pl.program_id(0)

    x = x_ref[...]
    x2_ref[...] = jnp.sum(x * x, axis=1, keepdims=True).reshape(R, 8, 1)
    xs_ref[...] = x * (-2.0)

    @pl.when(n == 0)
    def _consts():
        w = w_ref[...]
        c2 = jnp.sum(w * w, axis=1)
        c2_ref[...] = jnp.broadcast_to(c2[None, :], (8, BK))
        ii_ref[...] = lax.broadcasted_iota(jnp.int32, (8, BK), 1).astype(
            jnp.float32)

    m = lax.dot_general(xs_ref[...], w_ref[...], (((1,), (1,)), ((), ())),
                        preferred_element_type=jnp.float32)
    m3 = m.reshape(R, 8, BK)
    d = x2_ref[...] + m3 + c2_ref[...][None]

    bmin = jnp.min(d, axis=2, keepdims=True)
    ii = ii_ref[...][None]
    bidx = jnp.min(jnp.where(d == bmin, ii, jnp.inf),
                   axis=2, keepdims=True)
    idx_ref[...] = bidx.astype(jnp.int32).reshape(BN)


def _tc_argmin(flat, weight):
    n = flat.shape[0]
    return pl.pallas_call(
        _argmin_body,
        grid=(n // BN,),
        in_specs=[
            pl.BlockSpec((BN, LATENT), lambda i: (i, 0)),
            pl.BlockSpec((BK, LATENT), lambda i: (0, 0)),
        ],
        out_specs=pl.BlockSpec((BN,), lambda i: (i,)),
        out_shape=jax.ShapeDtypeStruct((n,), jnp.int32),
        scratch_shapes=[
            pltpu.VMEM((R, 8, 1), jnp.float32),
            pltpu.VMEM((8, BK), jnp.float32),
            pltpu.VMEM((8, BK), jnp.float32),
            pltpu.VMEM((BN, LATENT), jnp.float32),
        ],
    )(flat, weight)


_NC, _NS = 2, 16
_NW = _NC * _NS


def _sc_gather(weight, idx):
    n = idx.shape[0]
    bpw = n // _NW

    @functools.partial(
        pl.kernel,
        mesh=plsc.VectorSubcoreMesh(core_axis_name="c", subcore_axis_name="s"),
        out_type=jax.ShapeDtypeStruct((n, LATENT), jnp.float32),
        scratch_types=[
            pltpu.VMEM((bpw,), jnp.int32),
            pltpu.VMEM((bpw, LATENT), jnp.float32),
            pltpu.SemaphoreType.DMA,
        ],
    )
    def gather_k(table_hbm, idx_hbm, out_hbm, idx_v, rows_v, sem):
        wid = lax.axis_index("s") * _NC + lax.axis_index("c")
        base = wid * bpw
        pltpu.sync_copy(idx_hbm.at[pl.ds(base, bpw)], idx_v)
        pltpu.async_copy(table_hbm.at[idx_v], rows_v, sem).wait()
        pltpu.sync_copy(rows_v, out_hbm.at[pl.ds(base, bpw)])

    return gather_k(weight, idx)


def kernel(x, weight):
    flat = x.reshape(-1, LATENT)
    idx = _tc_argmin(flat, weight)
    codes = _sc_gather(weight, idx)
    return codes.reshape(x.shape)
